# baseline (speedup 1.0000x reference)
import functools

import jax
import jax.numpy as jnp
from jax import lax
from jax.experimental import pallas as pl
from jax.experimental.pallas import tpu as pltpu
from jax.experimental.pallas import tpu_sc as plsc

NC = 2
NS = 16
NW = NC * NS
SCHUNK = 128
LANES = 16


def _make_sc_degree(n_pad, e_pad):
    ept = e_pad // NW
    nchunks = ept // SCHUNK
    mesh = plsc.VectorSubcoreMesh(core_axis_name="c", subcore_axis_name="s")

    @functools.partial(
        pl.kernel,
        out_type=jax.ShapeDtypeStruct((NW * n_pad,), jnp.float32),
        mesh=mesh,
        compiler_params=pltpu.CompilerParams(needs_layout_passes=False),
        scratch_types=[
            pltpu.VMEM((n_pad,), jnp.float32),
            pltpu.VMEM((ept,), jnp.int32),
        ],
    )
    def deg_kernel(dstp_hbm, z1_hbm, out_hbm, hist, idx_d):
        cid = lax.axis_index("c")
        sid = lax.axis_index("s")
        wid = sid * NC + cid
        pltpu.sync_copy(z1_hbm, hist)
        pltpu.sync_copy(dstp_hbm.at[pl.ds(wid * ept, ept)], idx_d)
        ones = jnp.ones((LANES,), jnp.float32)

        def body(c, carry):
            for j in range(SCHUNK // LANES):
                d = idx_d[pl.ds(c * SCHUNK + j * LANES, LANES)]
                plsc.addupdate_scatter(hist, [d], ones)
            return carry

        lax.fori_loop(0, nchunks, body, 0)
        pltpu.sync_copy(hist, out_hbm.at[pl.ds(wid * n_pad, n_pad)])

    return deg_kernel


def _make_sc_scatter(n, n_pad, e_pad, h):
    ept = e_pad // NW
    nchunks = ept // SCHUNK
    stripe = n_pad // NS
    stripe_chunks = stripe // SCHUNK
    mesh = plsc.VectorSubcoreMesh(core_axis_name="c", subcore_axis_name="s")

    @functools.partial(
        pl.kernel,
        out_type=jax.ShapeDtypeStruct((NC * n_pad, h), jnp.float32),
        mesh=mesh,
        compiler_params=pltpu.CompilerParams(needs_layout_passes=False),
        scratch_types=[
            pltpu.VMEM((SCHUNK,), jnp.int32),
            pltpu.VMEM((SCHUNK,), jnp.int32),
            pltpu.VMEM((SCHUNK, h), jnp.float32),
            pltpu.VMEM_SHARED((n_pad, h), jnp.float32),
            pltpu.SemaphoreType.DMA,
        ],
    )
    def scatter_kernel(m_hbm, srcp_hbm, dstp_hbm, z_hbm, out_hbm,
                       idx_s, idx_d, rows, acc, sem):
        cid = lax.axis_index("c")
        sid = lax.axis_index("s")
        wid = sid * NC + cid
        pltpu.sync_copy(z_hbm, rows)
        for k in range(stripe_chunks):
            pltpu.sync_copy(rows, acc.at[pl.ds(sid * stripe + k * SCHUNK, SCHUNK)])
        plsc.subcore_barrier()
        base = wid * ept

        def body(c, carry):
            eb = base + c * SCHUNK
            pltpu.sync_copy(srcp_hbm.at[pl.ds(eb, SCHUNK)], idx_s)
            pltpu.sync_copy(dstp_hbm.at[pl.ds(eb, SCHUNK)], idx_d)
            pltpu.async_copy(m_hbm.at[idx_s], rows, sem).wait()
            pltpu.sync_copy(rows, acc.at[idx_d], add=True)
            return carry

        lax.fori_loop(0, nchunks, body, 0)
        plsc.subcore_barrier()
        for k in range(stripe_chunks):
            r0_ = sid * stripe + k * SCHUNK
            pltpu.sync_copy(acc.at[pl.ds(r0_, SCHUNK)],
                            out_hbm.at[pl.ds(cid * n_pad + r0_, SCHUNK)])

    return scatter_kernel


def _tc_dinv(degp):
    nw, nr, w = degp.shape

    def body(deg_ref, out_ref):
        out_ref[...] = lax.rsqrt(jnp.sum(deg_ref[...], axis=0) + 1.0)

    return pl.pallas_call(
        body, out_shape=jax.ShapeDtypeStruct((nr, w), jnp.float32))(degp)


def _tc_scale_matmul(x, w, dinv, blk):
    n, d = x.shape
    h = w.shape[1]
    grid = n // blk

    def body(x_ref, w_ref, s_ref, out_ref):
        out_ref[...] = s_ref[...] * jnp.dot(
            x_ref[...], w_ref[...], preferred_element_type=jnp.float32)

    return pl.pallas_call(
        body,
        grid=(grid,),
        in_specs=[
            pl.BlockSpec((blk, d), lambda i: (i, 0)),
            pl.BlockSpec((d, h), lambda i: (0, 0)),
            pl.BlockSpec((blk, 1), lambda i: (i, 0)),
        ],
        out_specs=pl.BlockSpec((blk, h), lambda i: (i, 0)),
        out_shape=jax.ShapeDtypeStruct((n, h), jnp.float32),
    )(x, w, dinv)


def _tc_post1(S, m, dinv, b, w2, blk):
    n, h = m.shape
    h2 = w2.shape[1]
    grid = n // blk

    def body(s_ref, m_ref, d_ref, b_ref, w_ref, out_ref):
        agg = s_ref[0] + s_ref[1] + m_ref[...]
        a = jnp.maximum(d_ref[...] * agg + b_ref[...], 0.0)
        out_ref[...] = d_ref[...] * jnp.dot(
            a, w_ref[...], preferred_element_type=jnp.float32)

    return pl.pallas_call(
        body,
        grid=(grid,),
        in_specs=[
            pl.BlockSpec((2, blk, h), lambda i: (0, i, 0)),
            pl.BlockSpec((blk, h), lambda i: (i, 0)),
            pl.BlockSpec((blk, 1), lambda i: (i, 0)),
            pl.BlockSpec((1, h), lambda i: (0, 0)),
            pl.BlockSpec((h, h2), lambda i: (0, 0)),
        ],
        out_specs=pl.BlockSpec((blk, h2), lambda i: (i, 0)),
        out_shape=jax.ShapeDtypeStruct((n, h2), jnp.float32),
    )(S, m, dinv, b, w2)


def _tc_post2(S, m, dinv, b, wlT, bl, blk):
    n, h = m.shape
    a_dim = wlT.shape[1]
    grid = n // blk

    def body(s_ref, m_ref, d_ref, b_ref, w_ref, bl_ref, out_ref, sacc, macc):
        i = pl.program_id(0)
        agg = s_ref[0] + s_ref[1] + m_ref[...]
        a = jnp.maximum(d_ref[...] * agg + b_ref[...], 0.0)
        bs = jnp.sum(a, axis=0, keepdims=True)
        bm = jnp.max(a, axis=0, keepdims=True)

        @pl.when(i == 0)
        def _():
            sacc[...] = bs
            macc[...] = bm

        @pl.when(i > 0)
        def _():
            sacc[...] = sacc[...] + bs
            macc[...] = jnp.maximum(macc[...], bm)

        @pl.when(i == grid - 1)
        def _():
            s = sacc[...]
            mx = macc[...]
            mean = s * (1.0 / n)
            out_ref[...] = (
                jnp.dot(mean, w_ref[0:h, :], preferred_element_type=jnp.float32)
                + jnp.dot(mx, w_ref[h:2 * h, :], preferred_element_type=jnp.float32)
                + jnp.dot(s, w_ref[2 * h:3 * h, :], preferred_element_type=jnp.float32)
                + bl_ref[...])

    return pl.pallas_call(
        body,
        grid=(grid,),
        in_specs=[
            pl.BlockSpec((2, blk, h), lambda i: (0, i, 0)),
            pl.BlockSpec((blk, h), lambda i: (i, 0)),
            pl.BlockSpec((blk, 1), lambda i: (i, 0)),
            pl.BlockSpec((1, h), lambda i: (0, 0)),
            pl.BlockSpec((3 * h, a_dim), lambda i: (0, 0)),
            pl.BlockSpec((1, a_dim), lambda i: (0, 0)),
        ],
        out_specs=pl.BlockSpec((1, a_dim), lambda i: (0, 0)),
        out_shape=jax.ShapeDtypeStruct((1, a_dim), jnp.float32),
        scratch_shapes=[
            pltpu.VMEM((1, h), jnp.float32),
            pltpu.VMEM((1, h), jnp.float32),
        ],
    )(S, m, dinv, b, wlT, bl)


def kernel(x, edge_index, pos, W1, b1, W2, b2, Wl, bl):
    n, d = x.shape
    h = W1.shape[1]
    e = edge_index.shape[1]
    gran = NS * SCHUNK
    n_pad = ((n + 1 + gran - 1) // gran) * gran
    egran = NW * SCHUNK * 2
    e_pad = ((e + egran - 1) // egran) * egran
    nchunks = e_pad // NW // SCHUNK

    src = edge_index[0]
    dst = edge_index[1]
    padn = e_pad - e
    srcp = jnp.concatenate([src, jnp.zeros((padn,), jnp.int32)])
    dstp = jnp.concatenate([dst, jnp.full((padn,), n, jnp.int32)])
    z1 = jnp.zeros((n_pad,), jnp.float32)
    z = jnp.zeros((SCHUNK, h), jnp.float32)

    degp = _make_sc_degree(n_pad, e_pad)(dstp, z1)
    dinv2 = _tc_dinv(degp.reshape(NW, n_pad // 128, 128))
    dinv = dinv2.reshape(-1)[:n].reshape(n, 1)

    blk = 1000 if n % 1000 == 0 else 8
    sc_scatter = _make_sc_scatter(n, n_pad, e_pad, h)

    m1 = _tc_scale_matmul(x, W1, dinv, blk)
    S1 = sc_scatter(m1, srcp, dstp, z).reshape(NC, n_pad, h)
    m2 = _tc_post1(S1, m1, dinv, b1.reshape(1, h), W2, blk)
    S2 = sc_scatter(m2, srcp, dstp, z).reshape(NC, n_pad, h)
    out = _tc_post2(S2, m2, dinv, b2.reshape(1, h), Wl.T, bl.reshape(1, -1), blk)
    return out


# speedup vs baseline: 1.2517x; 1.2517x over previous
import functools

import jax
import jax.numpy as jnp
from jax import lax
from jax.experimental import pallas as pl
from jax.experimental.pallas import tpu as pltpu
from jax.experimental.pallas import tpu_sc as plsc

NC = 2
NS = 16
NW = NC * NS
SCHUNK = 128
LANES = 16


def _make_sc_degree(n_pad, e_pad):
    ept = e_pad // NW
    nchunks = ept // SCHUNK
    mesh = plsc.VectorSubcoreMesh(core_axis_name="c", subcore_axis_name="s")

    @functools.partial(
        pl.kernel,
        out_type=jax.ShapeDtypeStruct((NW * n_pad,), jnp.float32),
        mesh=mesh,
        compiler_params=pltpu.CompilerParams(needs_layout_passes=False),
        scratch_types=[
            pltpu.VMEM((n_pad,), jnp.float32),
            pltpu.VMEM((SCHUNK,), jnp.int32),
        ],
    )
    def deg_kernel(dstp_hbm, z1_hbm, out_hbm, hist, idx_d):
        cid = lax.axis_index("c")
        sid = lax.axis_index("s")
        wid = sid * NC + cid
        pltpu.sync_copy(z1_hbm, hist)
        base = wid * ept
        ones = jnp.ones((LANES,), jnp.float32)

        def body(c, carry):
            pltpu.sync_copy(dstp_hbm.at[pl.ds(base + c * SCHUNK, SCHUNK)], idx_d)
            for j in range(SCHUNK // LANES):
                d = idx_d[pl.ds(j * LANES, LANES)]
                plsc.addupdate_scatter(hist, [d], ones)
            return carry

        lax.fori_loop(0, nchunks, body, 0)
        pltpu.sync_copy(hist, out_hbm.at[pl.ds(wid * n_pad, n_pad)])

    return deg_kernel


def _make_sc_scatter(n, n_pad, e_pad, h):
    ept = e_pad // NW
    nchunks = ept // SCHUNK
    stripe = n_pad // NS
    stripe_chunks = stripe // SCHUNK
    mesh = plsc.VectorSubcoreMesh(core_axis_name="c", subcore_axis_name="s")

    @functools.partial(
        pl.kernel,
        out_type=jax.ShapeDtypeStruct((NC * n_pad, h), jnp.float32),
        mesh=mesh,
        compiler_params=pltpu.CompilerParams(needs_layout_passes=False),
        scratch_types=[
            pltpu.VMEM((SCHUNK,), jnp.int32),
            pltpu.VMEM((SCHUNK,), jnp.int32),
            pltpu.VMEM((SCHUNK, h), jnp.float32),
            pltpu.VMEM_SHARED((n_pad, h), jnp.float32),
            pltpu.SemaphoreType.DMA,
        ],
    )
    def scatter_kernel(m_hbm, srcp_hbm, dstp_hbm, z_hbm, out_hbm,
                       idx_s, idx_d, rows, acc, sem):
        cid = lax.axis_index("c")
        sid = lax.axis_index("s")
        wid = sid * NC + cid
        pltpu.sync_copy(z_hbm, rows)
        for k in range(stripe_chunks):
            pltpu.sync_copy(rows, acc.at[pl.ds(sid * stripe + k * SCHUNK, SCHUNK)])
        plsc.subcore_barrier()
        base = wid * ept

        def body(c, carry):
            eb = base + c * SCHUNK
            pltpu.sync_copy(srcp_hbm.at[pl.ds(eb, SCHUNK)], idx_s)
            pltpu.sync_copy(dstp_hbm.at[pl.ds(eb, SCHUNK)], idx_d)
            pltpu.async_copy(m_hbm.at[idx_s], rows, sem).wait()
            pltpu.sync_copy(rows, acc.at[idx_d], add=True)
            return carry

        lax.fori_loop(0, nchunks, body, 0)
        plsc.subcore_barrier()
        for k in range(stripe_chunks):
            r0_ = sid * stripe + k * SCHUNK
            pltpu.sync_copy(acc.at[pl.ds(r0_, SCHUNK)],
                            out_hbm.at[pl.ds(cid * n_pad + r0_, SCHUNK)])

    return scatter_kernel


def _tc_dinv(degp):
    nw, nr, w = degp.shape

    def body(deg_ref, out_ref):
        out_ref[...] = lax.rsqrt(jnp.sum(deg_ref[...], axis=0) + 1.0)

    return pl.pallas_call(
        body, out_shape=jax.ShapeDtypeStruct((nr, w), jnp.float32))(degp)


def _tc_scale_matmul(x, w, dinv, blk):
    n, d = x.shape
    h = w.shape[1]
    grid = n // blk

    def body(x_ref, w_ref, s_ref, out_ref):
        out_ref[...] = s_ref[...] * jnp.dot(
            x_ref[...], w_ref[...], preferred_element_type=jnp.float32)

    return pl.pallas_call(
        body,
        grid=(grid,),
        in_specs=[
            pl.BlockSpec((blk, d), lambda i: (i, 0)),
            pl.BlockSpec((d, h), lambda i: (0, 0)),
            pl.BlockSpec((blk, 1), lambda i: (i, 0)),
        ],
        out_specs=pl.BlockSpec((blk, h), lambda i: (i, 0)),
        out_shape=jax.ShapeDtypeStruct((n, h), jnp.float32),
    )(x, w, dinv)


def _tc_post1(S, m, dinv, b, w2, blk):
    n, h = m.shape
    h2 = w2.shape[1]
    grid = n // blk

    def body(s_ref, m_ref, d_ref, b_ref, w_ref, out_ref):
        agg = s_ref[0] + s_ref[1] + m_ref[...]
        a = jnp.maximum(d_ref[...] * agg + b_ref[...], 0.0)
        out_ref[...] = d_ref[...] * jnp.dot(
            a, w_ref[...], preferred_element_type=jnp.float32)

    return pl.pallas_call(
        body,
        grid=(grid,),
        in_specs=[
            pl.BlockSpec((2, blk, h), lambda i: (0, i, 0)),
            pl.BlockSpec((blk, h), lambda i: (i, 0)),
            pl.BlockSpec((blk, 1), lambda i: (i, 0)),
            pl.BlockSpec((1, h), lambda i: (0, 0)),
            pl.BlockSpec((h, h2), lambda i: (0, 0)),
        ],
        out_specs=pl.BlockSpec((blk, h2), lambda i: (i, 0)),
        out_shape=jax.ShapeDtypeStruct((n, h2), jnp.float32),
    )(S, m, dinv, b, w2)


def _tc_post2(S, m, dinv, b, wlT, bl, blk):
    n, h = m.shape
    a_dim = wlT.shape[1]
    grid = n // blk

    def body(s_ref, m_ref, d_ref, b_ref, w_ref, bl_ref, out_ref, sacc, macc):
        i = pl.program_id(0)
        agg = s_ref[0] + s_ref[1] + m_ref[...]
        a = jnp.maximum(d_ref[...] * agg + b_ref[...], 0.0)
        bs = jnp.sum(a, axis=0, keepdims=True)
        bm = jnp.max(a, axis=0, keepdims=True)

        @pl.when(i == 0)
        def _():
            sacc[...] = bs
            macc[...] = bm

        @pl.when(i > 0)
        def _():
            sacc[...] = sacc[...] + bs
            macc[...] = jnp.maximum(macc[...], bm)

        @pl.when(i == grid - 1)
        def _():
            s = sacc[...]
            mx = macc[...]
            mean = s * (1.0 / n)
            out_ref[...] = (
                jnp.dot(mean, w_ref[0:h, :], preferred_element_type=jnp.float32)
                + jnp.dot(mx, w_ref[h:2 * h, :], preferred_element_type=jnp.float32)
                + jnp.dot(s, w_ref[2 * h:3 * h, :], preferred_element_type=jnp.float32)
                + bl_ref[...])

    return pl.pallas_call(
        body,
        grid=(grid,),
        in_specs=[
            pl.BlockSpec((2, blk, h), lambda i: (0, i, 0)),
            pl.BlockSpec((blk, h), lambda i: (i, 0)),
            pl.BlockSpec((blk, 1), lambda i: (i, 0)),
            pl.BlockSpec((1, h), lambda i: (0, 0)),
            pl.BlockSpec((3 * h, a_dim), lambda i: (0, 0)),
            pl.BlockSpec((1, a_dim), lambda i: (0, 0)),
        ],
        out_specs=pl.BlockSpec((1, a_dim), lambda i: (0, 0)),
        out_shape=jax.ShapeDtypeStruct((1, a_dim), jnp.float32),
        scratch_shapes=[
            pltpu.VMEM((1, h), jnp.float32),
            pltpu.VMEM((1, h), jnp.float32),
        ],
    )(S, m, dinv, b, wlT, bl)


def kernel(x, edge_index, pos, W1, b1, W2, b2, Wl, bl):
    n, d = x.shape
    h = W1.shape[1]
    e = edge_index.shape[1]
    gran = NS * SCHUNK
    n_pad = ((n + 1 + gran - 1) // gran) * gran
    egran = NW * SCHUNK
    e_pad = ((e + egran - 1) // egran) * egran
    nchunks = e_pad // NW // SCHUNK

    src = edge_index[0]
    dst = edge_index[1]
    padn = e_pad - e
    srcp = jnp.concatenate([src, jnp.zeros((padn,), jnp.int32)])
    dstp = jnp.concatenate([dst, jnp.full((padn,), n, jnp.int32)])
    z1 = jnp.zeros((n_pad,), jnp.float32)
    z = jnp.zeros((SCHUNK, h), jnp.float32)

    degp = _make_sc_degree(n_pad, e_pad)(dstp, z1)
    dinv2 = _tc_dinv(degp.reshape(NW, n_pad // 128, 128))
    dinv = dinv2.reshape(-1)[:n].reshape(n, 1)

    blk = 1000 if n % 1000 == 0 else 8
    sc_scatter = _make_sc_scatter(n, n_pad, e_pad, h)

    m1 = _tc_scale_matmul(x, W1, dinv, blk)
    S1 = sc_scatter(m1, srcp, dstp, z).reshape(NC, n_pad, h)
    m2 = _tc_post1(S1, m1, dinv, b1.reshape(1, h), W2, blk)
    S2 = sc_scatter(m2, srcp, dstp, z).reshape(NC, n_pad, h)
    out = _tc_post2(S2, m2, dinv, b2.reshape(1, h), Wl.T, bl.reshape(1, -1), blk)
    return out


# speedup vs baseline: 1.4043x; 1.1220x over previous
import functools

import jax
import jax.numpy as jnp
from jax import lax
from jax.experimental import pallas as pl
from jax.experimental.pallas import tpu as pltpu
from jax.experimental.pallas import tpu_sc as plsc

NC = 2
NS = 16
NW = NC * NS
SCHUNK = 128
LANES = 16


def _make_sc_degree(n_pad, e_pad):
    ept = e_pad // NW
    nchunks = ept // SCHUNK
    mesh = plsc.VectorSubcoreMesh(core_axis_name="c", subcore_axis_name="s")

    @functools.partial(
        pl.kernel,
        out_type=jax.ShapeDtypeStruct((NW * n_pad,), jnp.float32),
        mesh=mesh,
        compiler_params=pltpu.CompilerParams(needs_layout_passes=False),
        scratch_types=[
            pltpu.VMEM((n_pad,), jnp.float32),
            pltpu.VMEM((SCHUNK,), jnp.int32),
        ],
    )
    def deg_kernel(dstp_hbm, z1_hbm, out_hbm, hist, idx_d):
        cid = lax.axis_index("c")
        sid = lax.axis_index("s")
        wid = sid * NC + cid
        pltpu.sync_copy(z1_hbm, hist)
        base = wid * ept
        ones = jnp.ones((LANES,), jnp.float32)

        def body(c, carry):
            pltpu.sync_copy(dstp_hbm.at[pl.ds(base + c * SCHUNK, SCHUNK)], idx_d)
            for j in range(SCHUNK // LANES):
                d = idx_d[pl.ds(j * LANES, LANES)]
                plsc.addupdate_scatter(hist, [d], ones)
            return carry

        lax.fori_loop(0, nchunks, body, 0)
        pltpu.sync_copy(hist, out_hbm.at[pl.ds(wid * n_pad, n_pad)])

    return deg_kernel


def _make_sc_scatter(n, n_pad, e_pad, h):
    ept = e_pad // NW
    nchunks = ept // SCHUNK
    stripe = n_pad // NS
    stripe_chunks = stripe // SCHUNK
    total_chunks = e_pad // SCHUNK
    per_tile = total_chunks // NS
    q0 = total_chunks * 5 // (8 * NS)
    q1 = per_tile - q0
    mesh = plsc.VectorSubcoreMesh(core_axis_name="c", subcore_axis_name="s")

    @functools.partial(
        pl.kernel,
        out_type=jax.ShapeDtypeStruct((NC * n_pad, h), jnp.float32),
        mesh=mesh,
        compiler_params=pltpu.CompilerParams(needs_layout_passes=False),
        scratch_types=[
            pltpu.VMEM((SCHUNK,), jnp.int32),
            pltpu.VMEM((SCHUNK,), jnp.int32),
            pltpu.VMEM((SCHUNK, h), jnp.float32),
            pltpu.VMEM_SHARED((n_pad, h), jnp.float32),
            pltpu.SemaphoreType.DMA,
        ],
    )
    def scatter_kernel(m_hbm, srcp_hbm, dstp_hbm, z_hbm, out_hbm,
                       idx_s, idx_d, rows, acc, sem):
        cid = lax.axis_index("c")
        sid = lax.axis_index("s")
        pltpu.sync_copy(z_hbm, rows)
        for k in range(stripe_chunks):
            pltpu.sync_copy(rows, acc.at[pl.ds(sid * stripe + k * SCHUNK, SCHUNK)])
        plsc.subcore_barrier()
        my_chunks = jnp.where(cid == 0, q0, q1)
        cbase = jnp.where(cid == 0, sid * q0, NS * q0 + sid * q1)
        base = cbase * SCHUNK

        def body(c, carry):
            eb = base + c * SCHUNK
            pltpu.sync_copy(srcp_hbm.at[pl.ds(eb, SCHUNK)], idx_s)
            pltpu.sync_copy(dstp_hbm.at[pl.ds(eb, SCHUNK)], idx_d)
            pltpu.async_copy(m_hbm.at[idx_s], rows, sem).wait()
            pltpu.sync_copy(rows, acc.at[idx_d], add=True)
            return carry

        lax.fori_loop(0, my_chunks, body, 0)
        plsc.subcore_barrier()
        for k in range(stripe_chunks):
            r0_ = sid * stripe + k * SCHUNK
            pltpu.sync_copy(acc.at[pl.ds(r0_, SCHUNK)],
                            out_hbm.at[pl.ds(cid * n_pad + r0_, SCHUNK)])

    return scatter_kernel


def _tc_dinv(degp):
    nw, nr, w = degp.shape

    def body(deg_ref, out_ref):
        out_ref[...] = lax.rsqrt(jnp.sum(deg_ref[...], axis=0) + 1.0)

    return pl.pallas_call(
        body, out_shape=jax.ShapeDtypeStruct((nr, w), jnp.float32))(degp)


def _tc_scale_matmul(x, w, dinv, blk):
    n, d = x.shape
    h = w.shape[1]
    grid = n // blk

    def body(x_ref, w_ref, s_ref, out_ref):
        out_ref[...] = s_ref[...] * jnp.dot(
            x_ref[...], w_ref[...], preferred_element_type=jnp.float32)

    return pl.pallas_call(
        body,
        grid=(grid,),
        in_specs=[
            pl.BlockSpec((blk, d), lambda i: (i, 0)),
            pl.BlockSpec((d, h), lambda i: (0, 0)),
            pl.BlockSpec((blk, 1), lambda i: (i, 0)),
        ],
        out_specs=pl.BlockSpec((blk, h), lambda i: (i, 0)),
        out_shape=jax.ShapeDtypeStruct((n, h), jnp.float32),
    )(x, w, dinv)


def _tc_post1(S, m, dinv, b, w2, blk):
    n, h = m.shape
    h2 = w2.shape[1]
    grid = n // blk

    def body(s_ref, m_ref, d_ref, b_ref, w_ref, out_ref):
        agg = s_ref[0] + s_ref[1] + m_ref[...]
        a = jnp.maximum(d_ref[...] * agg + b_ref[...], 0.0)
        out_ref[...] = d_ref[...] * jnp.dot(
            a, w_ref[...], preferred_element_type=jnp.float32)

    return pl.pallas_call(
        body,
        grid=(grid,),
        in_specs=[
            pl.BlockSpec((2, blk, h), lambda i: (0, i, 0)),
            pl.BlockSpec((blk, h), lambda i: (i, 0)),
            pl.BlockSpec((blk, 1), lambda i: (i, 0)),
            pl.BlockSpec((1, h), lambda i: (0, 0)),
            pl.BlockSpec((h, h2), lambda i: (0, 0)),
        ],
        out_specs=pl.BlockSpec((blk, h2), lambda i: (i, 0)),
        out_shape=jax.ShapeDtypeStruct((n, h2), jnp.float32),
    )(S, m, dinv, b, w2)


def _tc_post2(S, m, dinv, b, wlT, bl, blk):
    n, h = m.shape
    a_dim = wlT.shape[1]
    grid = n // blk

    def body(s_ref, m_ref, d_ref, b_ref, w_ref, bl_ref, out_ref, sacc, macc):
        i = pl.program_id(0)
        agg = s_ref[0] + s_ref[1] + m_ref[...]
        a = jnp.maximum(d_ref[...] * agg + b_ref[...], 0.0)
        bs = jnp.sum(a, axis=0, keepdims=True)
        bm = jnp.max(a, axis=0, keepdims=True)

        @pl.when(i == 0)
        def _():
            sacc[...] = bs
            macc[...] = bm

        @pl.when(i > 0)
        def _():
            sacc[...] = sacc[...] + bs
            macc[...] = jnp.maximum(macc[...], bm)

        @pl.when(i == grid - 1)
        def _():
            s = sacc[...]
            mx = macc[...]
            mean = s * (1.0 / n)
            out_ref[...] = (
                jnp.dot(mean, w_ref[0:h, :], preferred_element_type=jnp.float32)
                + jnp.dot(mx, w_ref[h:2 * h, :], preferred_element_type=jnp.float32)
                + jnp.dot(s, w_ref[2 * h:3 * h, :], preferred_element_type=jnp.float32)
                + bl_ref[...])

    return pl.pallas_call(
        body,
        grid=(grid,),
        in_specs=[
            pl.BlockSpec((2, blk, h), lambda i: (0, i, 0)),
            pl.BlockSpec((blk, h), lambda i: (i, 0)),
            pl.BlockSpec((blk, 1), lambda i: (i, 0)),
            pl.BlockSpec((1, h), lambda i: (0, 0)),
            pl.BlockSpec((3 * h, a_dim), lambda i: (0, 0)),
            pl.BlockSpec((1, a_dim), lambda i: (0, 0)),
        ],
        out_specs=pl.BlockSpec((1, a_dim), lambda i: (0, 0)),
        out_shape=jax.ShapeDtypeStruct((1, a_dim), jnp.float32),
        scratch_shapes=[
            pltpu.VMEM((1, h), jnp.float32),
            pltpu.VMEM((1, h), jnp.float32),
        ],
    )(S, m, dinv, b, wlT, bl)


def kernel(x, edge_index, pos, W1, b1, W2, b2, Wl, bl):
    n, d = x.shape
    h = W1.shape[1]
    e = edge_index.shape[1]
    gran = NS * SCHUNK
    n_pad = ((n + 1 + gran - 1) // gran) * gran
    egran = NW * SCHUNK
    e_pad = ((e + egran - 1) // egran) * egran
    nchunks = e_pad // NW // SCHUNK

    src = edge_index[0]
    dst = edge_index[1]
    padn = e_pad - e
    srcp = jnp.concatenate([src, jnp.zeros((padn,), jnp.int32)])
    dstp = jnp.concatenate([dst, jnp.full((padn,), n, jnp.int32)])
    z1 = jnp.zeros((n_pad,), jnp.float32)
    z = jnp.zeros((SCHUNK, h), jnp.float32)

    degp = _make_sc_degree(n_pad, e_pad)(dstp, z1)
    dinv2 = _tc_dinv(degp.reshape(NW, n_pad // 128, 128))
    dinv = dinv2.reshape(-1)[:n].reshape(n, 1)

    blk = 1000 if n % 1000 == 0 else 8
    sc_scatter = _make_sc_scatter(n, n_pad, e_pad, h)

    m1 = _tc_scale_matmul(x, W1, dinv, blk)
    S1 = sc_scatter(m1, srcp, dstp, z).reshape(NC, n_pad, h)
    m2 = _tc_post1(S1, m1, dinv, b1.reshape(1, h), W2, blk)
    S2 = sc_scatter(m2, srcp, dstp, z).reshape(NC, n_pad, h)
    out = _tc_post2(S2, m2, dinv, b2.reshape(1, h), Wl.T, bl.reshape(1, -1), blk)
    return out


# speedup vs baseline: 1.4303x; 1.0185x over previous
import functools

import jax
import jax.numpy as jnp
from jax import lax
from jax.experimental import pallas as pl
from jax.experimental.pallas import tpu as pltpu
from jax.experimental.pallas import tpu_sc as plsc

NC = 2
NS = 16
NW = NC * NS
SCHUNK = 128
LANES = 16


def _make_sc_degree(n_pad, e_pad):
    ept = e_pad // NW
    nchunks = ept // SCHUNK
    mesh = plsc.VectorSubcoreMesh(core_axis_name="c", subcore_axis_name="s")

    @functools.partial(
        pl.kernel,
        out_type=jax.ShapeDtypeStruct((NW * n_pad,), jnp.float32),
        mesh=mesh,
        compiler_params=pltpu.CompilerParams(needs_layout_passes=False),
        scratch_types=[
            pltpu.VMEM((n_pad,), jnp.float32),
            pltpu.VMEM((SCHUNK,), jnp.int32),
        ],
    )
    def deg_kernel(dstp_hbm, z1_hbm, out_hbm, hist, idx_d):
        cid = lax.axis_index("c")
        sid = lax.axis_index("s")
        wid = sid * NC + cid
        pltpu.sync_copy(z1_hbm, hist)
        base = wid * ept
        ones = jnp.ones((LANES,), jnp.float32)

        def body(c, carry):
            pltpu.sync_copy(dstp_hbm.at[pl.ds(base + c * SCHUNK, SCHUNK)], idx_d)
            for j in range(SCHUNK // LANES):
                d = idx_d[pl.ds(j * LANES, LANES)]
                plsc.addupdate_scatter(hist, [d], ones)
            return carry

        lax.fori_loop(0, nchunks, body, 0)
        pltpu.sync_copy(hist, out_hbm.at[pl.ds(wid * n_pad, n_pad)])

    return deg_kernel


def _make_sc_scatter(n, n_pad, e_pad, h):
    ept = e_pad // NW
    nchunks = ept // SCHUNK
    stripe = n_pad // NS
    stripe_chunks = stripe // SCHUNK
    total_chunks = e_pad // SCHUNK
    per_tile = total_chunks // NS
    q0 = total_chunks * 11 // (16 * NS)
    q1 = per_tile - q0
    mesh = plsc.VectorSubcoreMesh(core_axis_name="c", subcore_axis_name="s")

    @functools.partial(
        pl.kernel,
        out_type=jax.ShapeDtypeStruct((NC * n_pad, h), jnp.float32),
        mesh=mesh,
        compiler_params=pltpu.CompilerParams(needs_layout_passes=False),
        scratch_types=[
            pltpu.VMEM((SCHUNK,), jnp.int32),
            pltpu.VMEM((SCHUNK,), jnp.int32),
            pltpu.VMEM((SCHUNK, h), jnp.float32),
            pltpu.VMEM_SHARED((n_pad, h), jnp.float32),
            pltpu.SemaphoreType.DMA,
        ],
    )
    def scatter_kernel(m_hbm, srcp_hbm, dstp_hbm, z_hbm, out_hbm,
                       idx_s, idx_d, rows, acc, sem):
        cid = lax.axis_index("c")
        sid = lax.axis_index("s")
        pltpu.sync_copy(z_hbm, rows)
        for k in range(stripe_chunks):
            pltpu.sync_copy(rows, acc.at[pl.ds(sid * stripe + k * SCHUNK, SCHUNK)])
        plsc.subcore_barrier()
        my_chunks = jnp.where(cid == 0, q0, q1)
        cbase = jnp.where(cid == 0, sid * q0, NS * q0 + sid * q1)
        base = cbase * SCHUNK

        def body(c, carry):
            eb = base + c * SCHUNK
            pltpu.sync_copy(srcp_hbm.at[pl.ds(eb, SCHUNK)], idx_s)
            pltpu.sync_copy(dstp_hbm.at[pl.ds(eb, SCHUNK)], idx_d)
            pltpu.async_copy(m_hbm.at[idx_s], rows, sem).wait()
            pltpu.sync_copy(rows, acc.at[idx_d], add=True)
            return carry

        lax.fori_loop(0, my_chunks, body, 0)
        plsc.subcore_barrier()
        for k in range(stripe_chunks):
            r0_ = sid * stripe + k * SCHUNK
            pltpu.sync_copy(acc.at[pl.ds(r0_, SCHUNK)],
                            out_hbm.at[pl.ds(cid * n_pad + r0_, SCHUNK)])

    return scatter_kernel


def _tc_dinv(degp):
    nw, nr, w = degp.shape

    def body(deg_ref, out_ref):
        out_ref[...] = lax.rsqrt(jnp.sum(deg_ref[...], axis=0) + 1.0)

    return pl.pallas_call(
        body, out_shape=jax.ShapeDtypeStruct((nr, w), jnp.float32))(degp)


def _tc_scale_matmul(x, w, dinv, blk):
    n, d = x.shape
    h = w.shape[1]
    grid = n // blk

    def body(x_ref, w_ref, s_ref, out_ref):
        out_ref[...] = s_ref[...] * jnp.dot(
            x_ref[...], w_ref[...], preferred_element_type=jnp.float32)

    return pl.pallas_call(
        body,
        grid=(grid,),
        in_specs=[
            pl.BlockSpec((blk, d), lambda i: (i, 0)),
            pl.BlockSpec((d, h), lambda i: (0, 0)),
            pl.BlockSpec((blk, 1), lambda i: (i, 0)),
        ],
        out_specs=pl.BlockSpec((blk, h), lambda i: (i, 0)),
        out_shape=jax.ShapeDtypeStruct((n, h), jnp.float32),
    )(x, w, dinv)


def _tc_post1(S, m, dinv, b, w2, blk):
    n, h = m.shape
    h2 = w2.shape[1]
    grid = n // blk

    def body(s_ref, m_ref, d_ref, b_ref, w_ref, out_ref):
        agg = s_ref[0] + s_ref[1] + m_ref[...]
        a = jnp.maximum(d_ref[...] * agg + b_ref[...], 0.0)
        out_ref[...] = d_ref[...] * jnp.dot(
            a, w_ref[...], preferred_element_type=jnp.float32)

    return pl.pallas_call(
        body,
        grid=(grid,),
        in_specs=[
            pl.BlockSpec((2, blk, h), lambda i: (0, i, 0)),
            pl.BlockSpec((blk, h), lambda i: (i, 0)),
            pl.BlockSpec((blk, 1), lambda i: (i, 0)),
            pl.BlockSpec((1, h), lambda i: (0, 0)),
            pl.BlockSpec((h, h2), lambda i: (0, 0)),
        ],
        out_specs=pl.BlockSpec((blk, h2), lambda i: (i, 0)),
        out_shape=jax.ShapeDtypeStruct((n, h2), jnp.float32),
    )(S, m, dinv, b, w2)


def _tc_post2(S, m, dinv, b, wlT, bl, blk):
    n, h = m.shape
    a_dim = wlT.shape[1]
    grid = n // blk

    def body(s_ref, m_ref, d_ref, b_ref, w_ref, bl_ref, out_ref, sacc, macc):
        i = pl.program_id(0)
        agg = s_ref[0] + s_ref[1] + m_ref[...]
        a = jnp.maximum(d_ref[...] * agg + b_ref[...], 0.0)
        bs = jnp.sum(a, axis=0, keepdims=True)
        bm = jnp.max(a, axis=0, keepdims=True)

        @pl.when(i == 0)
        def _():
            sacc[...] = bs
            macc[...] = bm

        @pl.when(i > 0)
        def _():
            sacc[...] = sacc[...] + bs
            macc[...] = jnp.maximum(macc[...], bm)

        @pl.when(i == grid - 1)
        def _():
            s = sacc[...]
            mx = macc[...]
            mean = s * (1.0 / n)
            out_ref[...] = (
                jnp.dot(mean, w_ref[0:h, :], preferred_element_type=jnp.float32)
                + jnp.dot(mx, w_ref[h:2 * h, :], preferred_element_type=jnp.float32)
                + jnp.dot(s, w_ref[2 * h:3 * h, :], preferred_element_type=jnp.float32)
                + bl_ref[...])

    return pl.pallas_call(
        body,
        grid=(grid,),
        in_specs=[
            pl.BlockSpec((2, blk, h), lambda i: (0, i, 0)),
            pl.BlockSpec((blk, h), lambda i: (i, 0)),
            pl.BlockSpec((blk, 1), lambda i: (i, 0)),
            pl.BlockSpec((1, h), lambda i: (0, 0)),
            pl.BlockSpec((3 * h, a_dim), lambda i: (0, 0)),
            pl.BlockSpec((1, a_dim), lambda i: (0, 0)),
        ],
        out_specs=pl.BlockSpec((1, a_dim), lambda i: (0, 0)),
        out_shape=jax.ShapeDtypeStruct((1, a_dim), jnp.float32),
        scratch_shapes=[
            pltpu.VMEM((1, h), jnp.float32),
            pltpu.VMEM((1, h), jnp.float32),
        ],
    )(S, m, dinv, b, wlT, bl)


def kernel(x, edge_index, pos, W1, b1, W2, b2, Wl, bl):
    n, d = x.shape
    h = W1.shape[1]
    e = edge_index.shape[1]
    gran = NS * SCHUNK
    n_pad = ((n + 1 + gran - 1) // gran) * gran
    egran = NW * SCHUNK
    e_pad = ((e + egran - 1) // egran) * egran
    nchunks = e_pad // NW // SCHUNK

    src = edge_index[0]
    dst = edge_index[1]
    padn = e_pad - e
    srcp = jnp.concatenate([src, jnp.zeros((padn,), jnp.int32)])
    dstp = jnp.concatenate([dst, jnp.full((padn,), n, jnp.int32)])
    z1 = jnp.zeros((n_pad,), jnp.float32)
    z = jnp.zeros((SCHUNK, h), jnp.float32)

    degp = _make_sc_degree(n_pad, e_pad)(dstp, z1)
    dinv2 = _tc_dinv(degp.reshape(NW, n_pad // 128, 128))
    dinv = dinv2.reshape(-1)[:n].reshape(n, 1)

    blk = 1000 if n % 1000 == 0 else 8
    sc_scatter = _make_sc_scatter(n, n_pad, e_pad, h)

    m1 = _tc_scale_matmul(x, W1, dinv, blk)
    S1 = sc_scatter(m1, srcp, dstp, z).reshape(NC, n_pad, h)
    m2 = _tc_post1(S1, m1, dinv, b1.reshape(1, h), W2, blk)
    S2 = sc_scatter(m2, srcp, dstp, z).reshape(NC, n_pad, h)
    out = _tc_post2(S2, m2, dinv, b2.reshape(1, h), Wl.T, bl.reshape(1, -1), blk)
    return out
